# baseline (speedup 1.0000x reference)
import jax
import jax.numpy as jnp
from jax.experimental import pallas as pl

N_E = 1024
E_DIM = 256
BETA = 0.25
TOK_TILE = 4096


def _vq_tile(zp_ref, zsq_ref, ct_ref, ct2_ref, csq_ref, zq_ref, idx_ref, part_ref):
    zp = zp_ref[...]
    zsq = zsq_ref[0]
    ct = ct_ref[...]
    ct2 = ct2_ref[...]
    csq = csq_ref[...]
    m2 = jnp.dot(zp, ct2)
    d = (zsq + csq) - m2
    mind = jnp.min(d, axis=1, keepdims=True)
    iota_k = jax.lax.broadcasted_iota(jnp.int32, d.shape, 1)
    big = jnp.int32(N_E)
    idx = jnp.min(jnp.where(d == mind, iota_k, big), axis=1, keepdims=True)
    onehot = jnp.where(iota_k == idx, 1.0, 0.0).astype(jnp.float32)
    nimg = TOK_TILE // 1024
    for i in range(nimg):
        oh = onehot[i * 1024:(i + 1) * 1024, :]
        zq_ref[i] = jax.lax.dot_general(ct, oh, (((1,), (1,)), ((), ())))
    idx_ref[...] = idx[None]
    part_ref[...] = jnp.sum(mind).reshape(1, 1, 1)


def kernel(z, codebook):
    B, C, H, W = z.shape
    ntok = B * H * W
    ntile = ntok // TOK_TILE
    zp = jnp.transpose(z, (0, 2, 3, 1)).reshape(ntok, E_DIM)
    ct = codebook.T
    ct2 = ct + ct
    csq = jnp.sum(codebook ** 2, axis=1).reshape(1, N_E)
    zsq3 = jnp.sum(zp ** 2, axis=1).reshape(ntile, TOK_TILE, 1)

    grid = (ntile,)
    zq_t, idx, parts = pl.pallas_call(
        _vq_tile,
        grid=grid,
        in_specs=[
            pl.BlockSpec((TOK_TILE, E_DIM), lambda b: (b, 0)),
            pl.BlockSpec((1, TOK_TILE, 1), lambda b: (b, 0, 0)),
            pl.BlockSpec((E_DIM, N_E), lambda b: (0, 0)),
            pl.BlockSpec((E_DIM, N_E), lambda b: (0, 0)),
            pl.BlockSpec((1, N_E), lambda b: (0, 0)),
        ],
        out_specs=[
            pl.BlockSpec((TOK_TILE // 1024, E_DIM, 1024), lambda b: (b, 0, 0)),
            pl.BlockSpec((1, TOK_TILE, 1), lambda b: (b, 0, 0)),
            pl.BlockSpec((1, 1, 1), lambda b: (b, 0, 0)),
        ],
        out_shape=[
            jax.ShapeDtypeStruct((B, E_DIM, 1024), jnp.float32),
            jax.ShapeDtypeStruct((ntile, TOK_TILE, 1), jnp.int32),
            jax.ShapeDtypeStruct((ntile, 1, 1), jnp.float32),
        ],
    )(zp, zsq3, ct, ct2, csq)

    z_q_out = zq_t.reshape(B, C, H, W)
    min_idx = idx.reshape(ntok)
    loss = (jnp.sum(parts) * ((1.0 + BETA) / float(ntok * E_DIM))).reshape(())
    return z_q_out, loss, min_idx


# speedup vs baseline: 1.2534x; 1.2534x over previous
import jax
import jax.numpy as jnp
from jax.experimental import pallas as pl
from jax.experimental.pallas import tpu as pltpu

N_E = 1024
E_DIM = 256
BETA = 0.25
TOK_TILE = 4096


def _vq_tile(zp_ref, ct2_ref, zq_ref, idx_ref, part_ref, csq_ref):
    zp = zp_ref[...]
    ct2 = ct2_ref[...]

    @pl.when(pl.program_id(0) == 0)
    def _():
        ch = 0.5 * ct2
        csq_ref[...] = jnp.sum(ch * ch, axis=0, keepdims=True)

    csq = csq_ref[...]
    m2 = jnp.dot(zp, ct2)
    zsq = jnp.sum(zp * zp, axis=1, keepdims=True)
    d = (zsq + csq) - m2
    mind = jnp.min(d, axis=1, keepdims=True)
    iota_k = jax.lax.broadcasted_iota(jnp.int32, d.shape, 1)
    big = jnp.int32(N_E)
    idx = jnp.min(jnp.where(d == mind, iota_k, big), axis=1, keepdims=True)
    onehot = jnp.where(iota_k == idx, 0.5, 0.0).astype(jnp.float32)
    nimg = TOK_TILE // 1024
    for i in range(nimg):
        oh = onehot[i * 1024:(i + 1) * 1024, :]
        zq_ref[i] = jax.lax.dot_general(ct2, oh, (((1,), (1,)), ((), ())))
    idx_ref[...] = idx[None]
    part_ref[...] = jnp.sum(mind).reshape(1, 1, 1)


def kernel(z, codebook):
    B, C, H, W = z.shape
    ntok = B * H * W
    ntile = ntok // TOK_TILE
    zp = jnp.transpose(z, (0, 2, 3, 1)).reshape(ntok, E_DIM)
    ct2 = codebook.T + codebook.T

    grid = (ntile,)
    zq_t, idx, parts = pl.pallas_call(
        _vq_tile,
        grid=grid,
        in_specs=[
            pl.BlockSpec((TOK_TILE, E_DIM), lambda b: (b, 0)),
            pl.BlockSpec((E_DIM, N_E), lambda b: (0, 0)),
        ],
        scratch_shapes=[pltpu.VMEM((1, N_E), jnp.float32)],
        out_specs=[
            pl.BlockSpec((TOK_TILE // 1024, E_DIM, 1024), lambda b: (b, 0, 0)),
            pl.BlockSpec((1, TOK_TILE, 1), lambda b: (b, 0, 0)),
            pl.BlockSpec((1, 1, 1), lambda b: (b, 0, 0)),
        ],
        out_shape=[
            jax.ShapeDtypeStruct((B, E_DIM, 1024), jnp.float32),
            jax.ShapeDtypeStruct((ntile, TOK_TILE, 1), jnp.int32),
            jax.ShapeDtypeStruct((ntile, 1, 1), jnp.float32),
        ],
    )(zp, ct2)

    z_q_out = zq_t.reshape(B, C, H, W)
    min_idx = idx.reshape(ntok)
    loss = (jnp.sum(parts) * ((1.0 + BETA) / float(ntok * E_DIM))).reshape(())
    return z_q_out, loss, min_idx


# speedup vs baseline: 1.2886x; 1.0281x over previous
import jax
import jax.numpy as jnp
from jax.experimental import pallas as pl
from jax.experimental.pallas import tpu as pltpu

N_E = 1024
E_DIM = 256
BETA = 0.25
TOK_TILE = 4096


def _vq_tile(zp_ref, cb_ref, zq_ref, idx_ref, part_ref, cb2_ref, csq_ref):
    zp = zp_ref[...]

    @pl.when(pl.program_id(0) == 0)
    def _():
        cb = cb_ref[...]
        cb2_ref[...] = cb + cb
        csq_ref[...] = jnp.sum(cb * cb, axis=1, keepdims=True).reshape(1, N_E)

    cb2 = cb2_ref[...]
    csq = csq_ref[...]
    m2 = jax.lax.dot_general(zp, cb2, (((1,), (1,)), ((), ())))
    zsq = jnp.sum(zp * zp, axis=1, keepdims=True)
    d = (zsq + csq) - m2
    mind = jnp.min(d, axis=1, keepdims=True)
    iota_k = jax.lax.broadcasted_iota(jnp.int32, d.shape, 1)
    big = jnp.int32(N_E)
    idx = jnp.min(jnp.where(d == mind, iota_k, big), axis=1, keepdims=True)
    onehot = jnp.where(iota_k == idx, 0.5, 0.0).astype(jnp.float32)
    nimg = TOK_TILE // 1024
    for i in range(nimg):
        oh = onehot[i * 1024:(i + 1) * 1024, :]
        zq_ref[i] = jax.lax.dot_general(cb2, oh, (((0,), (1,)), ((), ())))

    idx_ref[...] = idx[None]
    part_ref[...] = jnp.sum(mind).reshape(1, 1, 1)


def kernel(z, codebook):
    B, C, H, W = z.shape
    ntok = B * H * W
    ntile = ntok // TOK_TILE
    zp = jnp.transpose(z, (0, 2, 3, 1)).reshape(ntok, E_DIM)

    grid = (ntile,)
    zq_t, idx, parts = pl.pallas_call(
        _vq_tile,
        grid=grid,
        in_specs=[
            pl.BlockSpec((TOK_TILE, E_DIM), lambda b: (b, 0)),
            pl.BlockSpec((N_E, E_DIM), lambda b: (0, 0)),
        ],
        scratch_shapes=[pltpu.VMEM((N_E, E_DIM), jnp.float32),
                        pltpu.VMEM((1, N_E), jnp.float32)],
        out_specs=[
            pl.BlockSpec((TOK_TILE // 1024, E_DIM, 1024), lambda b: (b, 0, 0)),
            pl.BlockSpec((1, TOK_TILE, 1), lambda b: (b, 0, 0)),
            pl.BlockSpec((1, 1, 1), lambda b: (b, 0, 0)),
        ],
        out_shape=[
            jax.ShapeDtypeStruct((B, E_DIM, 1024), jnp.float32),
            jax.ShapeDtypeStruct((ntile, TOK_TILE, 1), jnp.int32),
            jax.ShapeDtypeStruct((ntile, 1, 1), jnp.float32),
        ],
    )(zp, codebook)

    z_q_out = zq_t.reshape(B, C, H, W)
    min_idx = idx.reshape(ntok)
    loss = (jnp.sum(parts) * ((1.0 + BETA) / float(ntok * E_DIM))).reshape(())
    return z_q_out, loss, min_idx
